# baseline (speedup 1.0000x reference)
import functools

import jax
import jax.numpy as jnp
import numpy as np
from jax import lax
from jax.experimental import pallas as pl
from jax.experimental.pallas import tpu as pltpu
from jax.experimental.pallas import tpu_sc as plsc

N = 10000
E = 160000
D = 256
DE = 16
L = 3

NC = 2
NS = 16
HALF = D // NC
CH = 80
NCHUNK = E // CH
CPS = NCHUNK // NS
N_PAD = 10240
ROWS_PER_SUB = N_PAD // NS
AGG_ROWS = N_PAD
RING = 3

_PERM = np.arange(HALF).reshape(HALF // 32, 2, 16).transpose(0, 2, 1).reshape(-1)


def _ee_body(ea_ref, we_ref, be_ref, out_ref):
    ea = ea_ref[...]
    for l in range(L):
        for c in range(NC):
            o = jnp.dot(ea, we_ref[l, c], preferred_element_type=jnp.float32)
            out_ref[l, c] = (o + be_ref[l, c][None, :]).astype(jnp.bfloat16)


def _ee_all(ea, We_r, be_r):
    BE = 2000
    return pl.pallas_call(
        _ee_body,
        grid=(E // BE,),
        in_specs=[
            pl.BlockSpec((BE, DE), lambda i: (i, 0)),
            pl.BlockSpec((L, NC, DE, HALF), lambda i: (0, 0, 0, 0)),
            pl.BlockSpec((L, NC, HALF), lambda i: (0, 0, 0)),
        ],
        out_specs=pl.BlockSpec((L, NC, BE, HALF), lambda i: (0, 0, i, 0)),
        out_shape=jax.ShapeDtypeStruct((L, NC, E, HALF), jnp.bfloat16),
    )(ea, We_r, be_r)


def _mm_body(agg_ref, wn_ref, bn_ref, out_ref):
    a0 = agg_ref[0]
    a1 = agg_ref[1]
    for c in range(NC):
        o = (jnp.dot(a0, wn_ref[0, c], preferred_element_type=jnp.float32)
             + jnp.dot(a1, wn_ref[1, c], preferred_element_type=jnp.float32)
             + bn_ref[c][None, :])
        out_ref[c] = jnp.where(o > 0, o, 0.01 * o)


def _mm_layer(agg, Wn_l, bn_l):
    BN = 1024
    return pl.pallas_call(
        _mm_body,
        grid=(N_PAD // BN,),
        in_specs=[
            pl.BlockSpec((NC, BN, HALF), lambda i: (0, i, 0)),
            pl.BlockSpec((NC, NC, HALF, HALF), lambda i: (0, 0, 0, 0)),
            pl.BlockSpec((NC, HALF), lambda i: (0, 0)),
        ],
        out_specs=pl.BlockSpec((NC, BN, HALF), lambda i: (0, i, 0)),
        out_shape=jax.ShapeDtypeStruct((NC, N_PAD, HALF), jnp.float32),
    )(agg, Wn_l, bn_l)


def _sc_layer(l, h_split, ee_all, src2d, dst2d):
    mesh = plsc.VectorSubcoreMesh(core_axis_name="c", subcore_axis_name="s")

    @functools.partial(
        pl.kernel,
        mesh=mesh,
        out_type=jax.ShapeDtypeStruct((NC, N_PAD, HALF), jnp.float32),
        scratch_types=(
            [pltpu.VMEM((CH, HALF), jnp.float32) for _ in range(RING)]
            + [pltpu.VMEM((CH * (HALF // 2),), jnp.int32) for _ in range(RING)]
            + [pltpu.VMEM((CH,), jnp.int32) for _ in range(2 * RING)]
            + [pltpu.VMEM_SHARED((AGG_ROWS, HALF), jnp.float32)]
            + [pltpu.SemaphoreType.DMA for _ in range(5 * RING)]
        ),
    )
    def k(h_hbm, ee_hbm, src_hbm, dst_hbm, out_hbm, *bufs):
        rows = bufs[0:RING]
        eev = bufs[RING:2 * RING]
        sidx = bufs[2 * RING:3 * RING]
        didx = bufs[3 * RING:4 * RING]
        agg_sh = bufs[4 * RING]
        sems = bufs[4 * RING + 1:]
        sg = sems[0:RING]
        se = sems[RING:2 * RING]
        ss = sems[2 * RING:3 * RING]
        ssi = sems[3 * RING:4 * RING]
        sdi = sems[4 * RING:5 * RING]
        c = lax.axis_index("c")
        s = lax.axis_index("s")
        base_r = s * ROWS_PER_SUB
        pltpu.sync_copy(h_hbm.at[c, pl.ds(base_r, ROWS_PER_SUB)],
                        agg_sh.at[pl.ds(base_r, ROWS_PER_SUB)])

        def issue_sidx(i, b):
            pltpu.async_copy(src_hbm.at[s * CPS + i], sidx[b], ssi[b])

        def wait_sidx(i, b):
            del i
            pltpu.make_async_copy(src_hbm.at[0], sidx[b], ssi[b]).wait()

        def issue_didx(i, b):
            pltpu.async_copy(dst_hbm.at[s * CPS + i], didx[b], sdi[b])

        def wait_didx(i, b):
            del i
            pltpu.make_async_copy(dst_hbm.at[0], didx[b], sdi[b]).wait()

        HW = CH * (HALF // 2)

        def ee_off(i):
            return pl.multiple_of(
                (l * NC + c) * (E * (HALF // 2)) + (s * CPS + i) * HW, HW)

        def issue_ge(i, b):
            pltpu.async_copy(h_hbm.at[c].at[sidx[b]], rows[b], sg[b])
            pltpu.async_copy(ee_hbm.at[pl.ds(ee_off(i), HW)],
                             eev[b], se[b])

        def wait_ge(i, b):
            pltpu.make_async_copy(h_hbm.at[c].at[sidx[b]],
                                  rows[b], sg[b]).wait()
            pltpu.make_async_copy(ee_hbm.at[pl.ds(ee_off(i), HW)],
                                  eev[b], se[b]).wait()

        plsc.subcore_barrier()

        for j in range(RING):
            issue_sidx(j, j)
            issue_didx(j, j)
        for j in range(2):
            wait_sidx(j, j)
            issue_ge(j, j)

        def body(t, carry):
            for b in range(RING):
                i = t * RING + b
                bp = (b + 2) % RING

                @pl.when(i < CPS)
                def _process():
                    wait_ge(i, b)

                    @pl.when(i + RING < CPS)
                    def _():
                        issue_sidx(i + RING, b)

                    def row_body(r, rc):
                        rbase = pl.multiple_of(r * (HALF // 2), HALF // 2)
                        for j in range(HALF // 32):
                            lo = pl.ds(32 * j, 16)
                            hi = pl.ds(32 * j + 16, 16)
                            w = eev[b][pl.ds(rbase + 16 * j, 16)]
                            ea = lax.bitcast_convert_type(
                                lax.shift_left(w, 16), jnp.float32)
                            eb = lax.bitcast_convert_type(
                                lax.bitwise_and(w, jnp.int32(-65536)),
                                jnp.float32)
                            rows[b][r, lo] = jnp.maximum(
                                rows[b][r, lo] + ea, 0.0)
                            rows[b][r, hi] = jnp.maximum(
                                rows[b][r, hi] + eb, 0.0)
                        return rc

                    lax.fori_loop(0, CH, row_body, 0)
                    wait_didx(i, b)
                    pltpu.async_copy(rows[b], agg_sh.at[didx[b]],
                                     ss[b], add=True)

                @pl.when((i >= 1) & (i - 1 < CPS))
                def _drain():
                    pltpu.make_async_copy(rows[bp], agg_sh.at[didx[bp]],
                                          ss[bp]).wait()

                @pl.when(i + 2 < CPS)
                def _prefetch():
                    @pl.when(i + 2 >= RING)
                    def _():
                        issue_didx(i + 2, bp)
                    wait_sidx(i + 2, bp)
                    issue_ge(i + 2, bp)
            return carry

        lax.fori_loop(0, (CPS + RING) // RING, body, 0)
        plsc.subcore_barrier()
        pltpu.sync_copy(agg_sh.at[pl.ds(base_r, ROWS_PER_SUB)],
                        out_hbm.at[c, pl.ds(base_r, ROWS_PER_SUB)])

    return k(h_split, ee_all, src2d, dst2d)


def kernel(x, edge_index, edge_attr, Wn, bn, We, be):
    src2d = edge_index[0].astype(jnp.int32).reshape(NCHUNK, CH)
    dst2d = edge_index[1].astype(jnp.int32).reshape(NCHUNK, CH)
    perm = jnp.asarray(_PERM)
    We_r = We.reshape(L, DE, NC, HALF).transpose(0, 2, 1, 3)[..., perm]
    be_r = be.reshape(L, NC, HALF)[..., perm]
    Wn_r = Wn.reshape(L, NC, HALF, NC, HALF).transpose(0, 1, 3, 2, 4)
    bn_r = bn.reshape(L, NC, HALF)
    x_p = jnp.concatenate([x, jnp.zeros((N_PAD - N, D), x.dtype)])
    h = x_p.reshape(N_PAD, NC, HALF).transpose(1, 0, 2)

    ee_bits = jax.lax.bitcast_convert_type(
        _ee_all(edge_attr, We_r, be_r).reshape(L, NC, E, HALF // 2, 2),
        jnp.int32).reshape(-1)

    for l in range(L):
        agg = _sc_layer(l, h, ee_bits, src2d, dst2d)
        h = _mm_layer(agg, Wn_r[l], bn_r[l])
    return h.transpose(1, 0, 2).reshape(N_PAD, D)[:N]
